# baseline (speedup 1.0000x reference)
import functools

import jax
import jax.numpy as jnp
import numpy as np
from jax import lax
from jax.experimental import pallas as pl
from jax.experimental.pallas import tpu as pltpu
from jax.experimental.pallas import tpu_sc as plsc

B = 4096
P = 2
L_SENT = 200
D = 64

_info = plsc.get_sparse_core_info()
_NC, _NS, _NL = _info.num_cores, _info.num_subcores, _info.num_lanes
_NW = _NC * _NS
ROWS = B * P
R_PER_W = ROWS // _NW
CHUNK = 128
N_CHUNK = R_PER_W // CHUNK


def _sc_gather(table, idx_raw):
    mesh = plsc.VectorSubcoreMesh(core_axis_name="c", subcore_axis_name="s")

    @functools.partial(
        pl.kernel,
        mesh=mesh,
        out_type=jax.ShapeDtypeStruct((ROWS, D), jnp.float32),
        compiler_params=pltpu.CompilerParams(use_tc_tiling_on_sc=False),
        scratch_types=[
            pltpu.VMEM((R_PER_W,), jnp.int32),
            pltpu.VMEM((N_CHUNK, CHUNK), jnp.int32),
            pltpu.VMEM((R_PER_W, D), jnp.float32),
            pltpu.SemaphoreType.DMA,
        ],
    )
    def k(table_hbm, idx_hbm, out_hbm, raw_v, gidx_v, rows_v, sem):
        wid = lax.axis_index("s") * _NC + lax.axis_index("c")
        base = wid * R_PER_W
        pltpu.sync_copy(idx_hbm.at[pl.ds(base, R_PER_W)], raw_v)
        per_chunk_vecs = CHUNK // _NL
        lane_vec = (lax.iota(jnp.int32, _NL) >> 1) * L_SENT
        for t in range(R_PER_W // _NL):
            s = ((base + t * _NL) // P) * L_SENT
            g = raw_v[pl.ds(t * _NL, _NL)] + lane_vec + s
            gidx_v[t // per_chunk_vecs,
                   pl.ds((t % per_chunk_vecs) * _NL, _NL)] = g
        copies = []
        for c in range(N_CHUNK):
            copies.append(
                pltpu.async_copy(
                    table_hbm.at[gidx_v.at[c]],
                    rows_v.at[pl.ds(c * CHUNK, CHUNK)],
                    sem,
                )
            )
        for cp in copies:
            cp.wait()
        pltpu.sync_copy(rows_v, out_hbm.at[pl.ds(base, R_PER_W)])

    return k(table, idx_raw)


def kernel(sentence_matrix, entity_pos_index):
    table = sentence_matrix.reshape(B * L_SENT, D)
    idx_raw = entity_pos_index.astype(jnp.int32).reshape(ROWS)
    out = _sc_gather(table, idx_raw)
    return out.reshape(B, P, D)


# speedup vs baseline: 2.7154x; 2.7154x over previous
import functools

import jax
import jax.numpy as jnp
from jax import lax
from jax.experimental import pallas as pl
from jax.experimental.pallas import tpu as pltpu
from jax.experimental.pallas import tpu_sc as plsc

B = 4096
P = 2
L_SENT = 200
D = 64

_info = plsc.get_sparse_core_info()
_NC, _NS, _NL = _info.num_cores, _info.num_subcores, _info.num_lanes
_NW = _NC * _NS
ROWS = B * P
R_PER_W = ROWS // _NW


def _sc_gather(table, idx_flat):
    mesh = plsc.VectorSubcoreMesh(core_axis_name="c", subcore_axis_name="s")

    @functools.partial(
        pl.kernel,
        mesh=mesh,
        out_type=jax.ShapeDtypeStruct((ROWS, D), jnp.float32),
        compiler_params=pltpu.CompilerParams(needs_layout_passes=False),
        scratch_types=[
            pltpu.VMEM((R_PER_W,), jnp.int32),
            pltpu.VMEM((R_PER_W, D), jnp.float32),
            pltpu.SemaphoreType.DMA,
        ],
    )
    def k(table_hbm, idx_hbm, out_hbm, raw_v, rows_v, sem):
        wid = lax.axis_index("s") * _NC + lax.axis_index("c")
        base = wid * R_PER_W
        pltpu.sync_copy(idx_hbm.at[pl.ds(base, R_PER_W)], raw_v)
        lane = lax.iota(jnp.int32, _NL)
        lane_vec = (lane >> 1) * L_SENT
        zero = jnp.zeros((_NL,), jnp.int32)
        for g in range(R_PER_W // _NL):
            s = ((base + g * _NL) >> 1) * L_SENT
            gid = raw_v[pl.ds(g * _NL, _NL)] + lane_vec + s
            for l in range(_NL):
                row = jnp.sum(jnp.where(lane == l, gid, zero))
                pltpu.make_async_copy(
                    table_hbm.at[pl.ds(row, 1)],
                    rows_v.at[pl.ds(g * _NL + l, 1)],
                    sem,
                ).start()
        pltpu.make_async_copy(
            table_hbm.at[pl.ds(0, R_PER_W)], rows_v, sem
        ).wait()
        pltpu.sync_copy(rows_v, out_hbm.at[pl.ds(base, R_PER_W)])

    return k(table, idx_flat)


def kernel(sentence_matrix, entity_pos_index):
    table = sentence_matrix.reshape(B * L_SENT, D)
    idx_flat = entity_pos_index.astype(jnp.int32).reshape(ROWS)
    out = _sc_gather(table, idx_flat)
    return out.reshape(B, P, D)
